# baseline (speedup 1.0000x reference)
import functools

import jax
import jax.numpy as jnp
from jax import lax
from jax.experimental import pallas as pl
from jax.experimental.pallas import tpu as pltpu
from jax.experimental.pallas import tpu_sc as plsc


def _make_gather(V, D, B):
    info = plsc.get_sparse_core_info()
    NC, NS = info.num_cores, info.num_subcores
    NW = NC * NS
    assert B % NW == 0
    b_per_w = B // NW
    CH = 512
    assert b_per_w % CH == 0
    n_chunks = b_per_w // CH
    mesh = plsc.VectorSubcoreMesh(core_axis_name="c", subcore_axis_name="s")

    @functools.partial(
        pl.kernel,
        mesh=mesh,
        out_type=jax.ShapeDtypeStruct((B, D), jnp.float32),
        scratch_types=[
            pltpu.VMEM((CH,), jnp.int32),
            pltpu.VMEM((CH, D), jnp.float32),
            pltpu.SemaphoreType.DMA,
        ],
        compiler_params=pltpu.CompilerParams(use_tc_tiling_on_sc=False),
    )
    def gather_kernel(table_hbm, idx_hbm, out_hbm, idx_v, rows_v, sem):
        wid = lax.axis_index("s") * NC + lax.axis_index("c")
        base = wid * b_per_w

        def body(i, carry):
            off = base + i * CH
            pltpu.sync_copy(idx_hbm.at[pl.ds(off, CH)], idx_v)
            pltpu.async_copy(table_hbm.at[idx_v], rows_v, sem).wait()
            pltpu.sync_copy(rows_v, out_hbm.at[pl.ds(off, CH)])
            return carry

        lax.fori_loop(0, n_chunks, body, 0)

    return gather_kernel


def kernel(x, weight):
    V, D = weight.shape
    orig_shape = x.shape
    flat = x.reshape(-1).astype(jnp.int32)
    B = flat.shape[0]
    out = _make_gather(V, D, B)(weight, flat)
    return out.reshape(*orig_shape, D)


# speedup vs baseline: 1.0455x; 1.0455x over previous
import functools

import jax
import jax.numpy as jnp
from jax import lax
from jax.experimental import pallas as pl
from jax.experimental.pallas import tpu as pltpu
from jax.experimental.pallas import tpu_sc as plsc


def _make_gather(V, D, B, CH):
    info = plsc.get_sparse_core_info()
    NC, NS = info.num_cores, info.num_subcores
    NW = NC * NS
    assert B % NW == 0
    b_per_w = B // NW
    assert b_per_w % CH == 0
    n = b_per_w // CH
    assert n >= 4 and n % 2 == 0
    mesh = plsc.VectorSubcoreMesh(core_axis_name="c", subcore_axis_name="s")

    @functools.partial(
        pl.kernel,
        mesh=mesh,
        out_type=jax.ShapeDtypeStruct((B, D), jnp.float32),
        scratch_types=[
            pltpu.VMEM((CH,), jnp.int32),
            pltpu.VMEM((CH,), jnp.int32),
            pltpu.VMEM((CH, D), jnp.float32),
            pltpu.VMEM((CH, D), jnp.float32),
            pltpu.SemaphoreType.DMA,
            pltpu.SemaphoreType.DMA,
            pltpu.SemaphoreType.DMA,
            pltpu.SemaphoreType.DMA,
            pltpu.SemaphoreType.DMA,
            pltpu.SemaphoreType.DMA,
        ],
        compiler_params=pltpu.CompilerParams(use_tc_tiling_on_sc=False),
    )
    def gather_kernel(table_hbm, idx_hbm, out_hbm,
                      ibuf0, ibuf1, rbuf0, rbuf1,
                      isem0, isem1, gsem0, gsem1, osem0, osem1):
        ibuf = (ibuf0, ibuf1)
        rbuf = (rbuf0, rbuf1)
        isem = (isem0, isem1)
        gsem = (gsem0, gsem1)
        osem = (osem0, osem1)
        wid = lax.axis_index("s") * NC + lax.axis_index("c")
        base = wid * b_per_w

        def idx_src(c):
            return idx_hbm.at[pl.ds(base + c * CH, CH)]

        def out_dst(c):
            return out_hbm.at[pl.ds(base + c * CH, CH)]

        i0 = pltpu.async_copy(idx_src(0), ibuf[0], isem[0])
        i1 = pltpu.async_copy(idx_src(1), ibuf[1], isem[1])
        i0.wait()
        g0 = pltpu.async_copy(table_hbm.at[ibuf[0]], rbuf[0], gsem[0])
        g0.wait()
        pltpu.async_copy(rbuf[0], out_dst(0), osem[0])
        i1.wait()
        pltpu.async_copy(table_hbm.at[ibuf[1]], rbuf[1], gsem[1])
        pltpu.async_copy(idx_src(2), ibuf[0], isem[0])

        def body(i, carry):
            for b in (0, 1):
                c = 2 + 2 * i + b
                o = b ^ 1
                pltpu.make_async_copy(rbuf[b], out_dst(0), osem[b]).wait()
                pltpu.make_async_copy(idx_src(0), ibuf[b], isem[b]).wait()
                pltpu.async_copy(table_hbm.at[ibuf[b]], rbuf[b], gsem[b])
                pltpu.make_async_copy(
                    table_hbm.at[ibuf[o]], rbuf[o], gsem[o]).wait()
                c_next = jnp.minimum(c + 1, n - 1)
                pltpu.async_copy(idx_src(c_next), ibuf[o], isem[o])
                pltpu.async_copy(rbuf[o], out_dst(c - 1), osem[o])
            return carry

        lax.fori_loop(0, (n - 2) // 2, body, 0, unroll=False)

        bl = (n - 1) % 2
        pltpu.make_async_copy(
            table_hbm.at[ibuf[bl]], rbuf[bl], gsem[bl]).wait()
        pltpu.async_copy(rbuf[bl], out_dst(n - 1), osem[bl])
        pltpu.make_async_copy(idx_src(0), ibuf[bl ^ 1], isem[bl ^ 1]).wait()
        pltpu.make_async_copy(rbuf[bl ^ 1], out_dst(0), osem[bl ^ 1]).wait()
        pltpu.make_async_copy(rbuf[bl], out_dst(0), osem[bl]).wait()

    return gather_kernel


def kernel(x, weight):
    V, D = weight.shape
    orig_shape = x.shape
    flat = x.reshape(-1).astype(jnp.int32)
    B = flat.shape[0]
    out = _make_gather(V, D, B, 800)(weight, flat)
    return out.reshape(*orig_shape, D)
